# baseline (speedup 1.0000x reference)
import functools

import jax
import jax.numpy as jnp
from jax import lax
from jax.experimental import pallas as pl
from jax.experimental.pallas import tpu as pltpu
from jax.experimental.pallas import tpu_sc as plsc

N = 10000
E = 320000
D = 128
HD = 64
G = 64
EPS = 1e-7

NC = 2
NS = 16
CH = 40
EPT = E // NS
NCHUNK = EPT // CH
SR = 640

_mesh = plsc.VectorSubcoreMesh(core_axis_name="c", subcore_axis_name="s")


BK = 20
NBK = NCHUNK // BK


@functools.partial(
    pl.kernel,
    out_type=jax.ShapeDtypeStruct((2 * N, D), jnp.float32),
    mesh=_mesh,
    scratch_types=[
        pltpu.VMEM((BK, CH), jnp.int32),
        pltpu.VMEM((BK, CH), jnp.int32),
        pltpu.VMEM((CH, D), jnp.float32),
        pltpu.VMEM((CH, D), jnp.float32),
        pltpu.VMEM((CH, D), jnp.float32),
        pltpu.VMEM((CH, D), jnp.float32),
        pltpu.VMEM((CH, D), jnp.float32),
        pltpu.VMEM_SHARED((N, D), jnp.float32),
        pltpu.SemaphoreType.DMA,
        pltpu.SemaphoreType.DMA,
    ],
)
def _sc_edge_pass(xs_hbm, es_hbm, src_hbm, dst_hbm, a_out,
                  srcb, dstb, xv0, xv1, ev0, ev1, pq, acc_sh, sg, se):
    c = lax.axis_index("c")
    s = lax.axis_index("s")
    coffx = c * N
    xvs, evs = (xv0, xv1), (ev0, ev1)

    def zrow(i, carry):
        for cc in range(D // 16):
            pq[i, pl.ds(cc * 16, 16)] = jnp.zeros((16,), jnp.float32)
        return carry
    lax.fori_loop(0, CH, zrow, 0)
    stripe0 = s * SR

    @pl.when(s < NS - 1)
    def _():
        for k in range(SR // CH):
            r0 = pl.multiple_of(stripe0 + k * CH, 8)
            pltpu.sync_copy(pq, acc_sh.at[pl.ds(r0, CH)])

    @pl.when(s == NS - 1)
    def _():
        for k in range((N - (NS - 1) * SR) // CH):
            r0 = pl.multiple_of(stripe0 + k * CH, 8)
            pltpu.sync_copy(pq, acc_sh.at[pl.ds(r0, CH)])

    plsc.subcore_barrier()

    e0 = s * EPT

    def _ebase(g):
        return pl.ds(pl.multiple_of(e0 + g * CH, 8), CH)

    def _compute_scatter(xv, ev, drow):
        def crow(r4, carry3):
            for rr in range(4):
                r = r4 * 4 + rr
                for cc in range(HD // 16):
                    sl = pl.ds(c * HD + cc * 16, 16)
                    m = jnp.maximum(xv[r, sl] + ev[r, sl], 0.0) + EPS
                    p = jnp.exp(m)
                    pq[r, pl.ds(cc * 16, 16)] = p
                    pq[r, pl.ds(HD + cc * 16, 16)] = p * m
            return carry3
        lax.fori_loop(0, CH // 4, crow, 0)
        pltpu.sync_copy(pq, acc_sh.at[drow], add=True)

    def block(j, carry):
        @pl.when(j >= 1)
        def _():
            _compute_scatter(xv1, ev1, dstb.at[BK - 1])

        pltpu.sync_copy(src_hbm.at[s, j], srcb)
        pltpu.sync_copy(dst_hbm.at[s, j], dstb)

        def pair(ci, carry2):
            for b in range(2):
                i_loc = ci * 2 + b
                g = j * BK + i_loc
                d1 = pltpu.async_copy(xs_hbm.at[srcb.at[i_loc]], xvs[b], sg)
                d2 = pltpu.async_copy(es_hbm.at[_ebase(g)], evs[b], se)

                @pl.when(i_loc >= 1)
                def _():
                    _compute_scatter(xvs[1 - b], evs[1 - b],
                                     dstb.at[i_loc - 1])

                d1.wait()
                d2.wait()
            return carry2
        lax.fori_loop(0, BK // 2, pair, 0)
        return carry
    lax.fori_loop(0, NBK, block, 0)

    _compute_scatter(xv1, ev1, dstb.at[BK - 1])
    plsc.subcore_barrier()

    @pl.when(s < NS - 1)
    def _():
        for k in range(SR // CH):
            r0 = pl.multiple_of(stripe0 + k * CH, 8)
            pltpu.sync_copy(acc_sh.at[pl.ds(r0, CH)],
                            a_out.at[pl.ds(pl.multiple_of(coffx + r0, 8), CH)])

    @pl.when(s == NS - 1)
    def _():
        for k in range((N - (NS - 1) * SR) // CH):
            r0 = pl.multiple_of(stripe0 + k * CH, 8)
            pltpu.sync_copy(acc_sh.at[pl.ds(r0, CH)],
                            a_out.at[pl.ds(pl.multiple_of(coffx + r0, 8), CH)])


BE = 4000
NEB = E // BE
BN = 2000
NB = N // BN


def _edge_mlp_body(ea_ref, w_ref, b_ref, o_ref):
    o_ref[...] = (
        jnp.dot(ea_ref[...], w_ref[...], preferred_element_type=jnp.float32)
        + b_ref[...]
    )


def _edge_mlp(edge_attr, W_edge, b_edge):
    return pl.pallas_call(
        _edge_mlp_body,
        grid=(NEB,),
        in_specs=[
            pl.BlockSpec((BE, 16), lambda g: (g, 0)),
            pl.BlockSpec((16, D), lambda g: (0, 0)),
            pl.BlockSpec((1, D), lambda g: (0, 0)),
        ],
        out_specs=pl.BlockSpec((BE, D), lambda g: (g, 0)),
        out_shape=jax.ShapeDtypeStruct((E, D), jnp.float32),
    )(edge_attr, W_edge, b_edge.reshape(1, D))


def _aggr_mlp1_body(alo_ref, ahi_ref, x_ref, w1_ref, h1_ref, sum_ref, sq_ref):
    alo = alo_ref[...]
    ahi = ahi_ref[...]
    aggr = jnp.concatenate(
        [alo[:, HD:] / (alo[:, :HD] + 1e-16), ahi[:, HD:] / (ahi[:, :HD] + 1e-16)],
        axis=1,
    )
    out = aggr + x_ref[...]
    h1 = jnp.dot(out, w1_ref[...], preferred_element_type=jnp.float32)
    h1_ref[...] = h1

    @pl.when(pl.program_id(0) == 0)
    def _():
        sum_ref[...] = jnp.zeros_like(sum_ref)
        sq_ref[...] = jnp.zeros_like(sq_ref)

    sum_ref[...] += jnp.sum(h1, 0, keepdims=True)
    sq_ref[...] += jnp.sum(h1 * h1, 0, keepdims=True)


def _aggr_mlp1(a, x, W1):
    return pl.pallas_call(
        _aggr_mlp1_body,
        grid=(NB,),
        in_specs=[
            pl.BlockSpec((BN, D), lambda g: (g, 0)),
            pl.BlockSpec((BN, D), lambda g: (NB + g, 0)),
            pl.BlockSpec((BN, D), lambda g: (g, 0)),
            pl.BlockSpec((D, 2 * D), lambda g: (0, 0)),
        ],
        out_specs=[
            pl.BlockSpec((BN, 2 * D), lambda g: (g, 0)),
            pl.BlockSpec((1, 2 * D), lambda g: (0, 0)),
            pl.BlockSpec((1, 2 * D), lambda g: (0, 0)),
        ],
        out_shape=[
            jax.ShapeDtypeStruct((N, 2 * D), jnp.float32),
            jax.ShapeDtypeStruct((1, 2 * D), jnp.float32),
            jax.ShapeDtypeStruct((1, 2 * D), jnp.float32),
        ],
    )(a, a, x, W1)


def _bn_mlp2_body(h1_ref, sum_ref, sq_ref, bs_ref, bb_ref, w2_ref,
                  h2_ref, sum2_ref, sq2_ref):
    mu = sum_ref[...] / N
    var = sq_ref[...] / N - mu * mu
    inv = lax.rsqrt(var + 1e-5) * bs_ref[...]
    hn = jnp.maximum((h1_ref[...] - mu) * inv + bb_ref[...], 0.0)
    h2 = jnp.dot(hn, w2_ref[...], preferred_element_type=jnp.float32)
    h2_ref[...] = h2

    @pl.when(pl.program_id(0) == 0)
    def _():
        sum2_ref[...] = jnp.zeros_like(sum2_ref)
        sq2_ref[...] = jnp.zeros_like(sq2_ref)

    sum2_ref[...] += jnp.sum(h2, 0, keepdims=True)
    sq2_ref[...] += jnp.sum(h2 * h2, 0, keepdims=True)


def _bn_mlp2(h1, s1, q1, bs, bb, W2):
    return pl.pallas_call(
        _bn_mlp2_body,
        grid=(NB,),
        in_specs=[
            pl.BlockSpec((BN, 2 * D), lambda g: (g, 0)),
            pl.BlockSpec((1, 2 * D), lambda g: (0, 0)),
            pl.BlockSpec((1, 2 * D), lambda g: (0, 0)),
            pl.BlockSpec((1, 2 * D), lambda g: (0, 0)),
            pl.BlockSpec((1, 2 * D), lambda g: (0, 0)),
            pl.BlockSpec((2 * D, D), lambda g: (0, 0)),
        ],
        out_specs=[
            pl.BlockSpec((BN, D), lambda g: (g, 0)),
            pl.BlockSpec((1, D), lambda g: (0, 0)),
            pl.BlockSpec((1, D), lambda g: (0, 0)),
        ],
        out_shape=[
            jax.ShapeDtypeStruct((N, D), jnp.float32),
            jax.ShapeDtypeStruct((1, D), jnp.float32),
            jax.ShapeDtypeStruct((1, D), jnp.float32),
        ],
    )(h1, s1, q1, bs.reshape(1, 2 * D), bb.reshape(1, 2 * D), W2)


def _bn_out_body(h2_ref, sum_ref, sq_ref, ns_ref, nb_ref, xn_ref):
    mu = sum_ref[...] / N
    var = sq_ref[...] / N - mu * mu
    inv = lax.rsqrt(var + 1e-5) * ns_ref[...]
    xn_ref[...] = jnp.maximum((h2_ref[...] - mu) * inv + nb_ref[...], 0.0)


def _bn_out(h2, s2, q2, ns, nb):
    return pl.pallas_call(
        _bn_out_body,
        grid=(NB,),
        in_specs=[
            pl.BlockSpec((BN, D), lambda g: (g, 0)),
            pl.BlockSpec((1, D), lambda g: (0, 0)),
            pl.BlockSpec((1, D), lambda g: (0, 0)),
            pl.BlockSpec((1, D), lambda g: (0, 0)),
            pl.BlockSpec((1, D), lambda g: (0, 0)),
        ],
        out_specs=pl.BlockSpec((BN, D), lambda g: (g, 0)),
        out_shape=jax.ShapeDtypeStruct((N, D), jnp.float32),
    )(h2, s2, q2, ns.reshape(1, D), nb.reshape(1, D))


BP = 80
NPB = N // BP


def _pool_body(b_ref, x_ref, wl_ref, bl_ref, o_ref, acc, cnt):
    g = pl.program_id(0)

    @pl.when(g == 0)
    def _():
        acc[...] = jnp.zeros_like(acc)
        cnt[...] = jnp.zeros_like(cnt)

    oh_t = (
        lax.broadcasted_iota(jnp.int32, (G, BP), 0) == b_ref[0, 0, :][None, :]
    ).astype(jnp.float32)
    acc[...] += jnp.dot(oh_t, x_ref[...], preferred_element_type=jnp.float32)
    cnt[...] += jnp.dot(oh_t, jnp.ones((BP, D), jnp.float32),
                        preferred_element_type=jnp.float32)

    @pl.when(g == NPB - 1)
    def _():
        pooled = acc[...] / jnp.maximum(cnt[...], 1.0)
        o_ref[...] = (
            jnp.dot(pooled, wl_ref[...], preferred_element_type=jnp.float32)
            + bl_ref[...]
        )


def _pool_linear(batch, h, W_lin, b_lin):
    return pl.pallas_call(
        _pool_body,
        grid=(NPB,),
        in_specs=[
            pl.BlockSpec((1, 1, BP), lambda g: (g, 0, 0)),
            pl.BlockSpec((BP, D), lambda g: (g, 0)),
            pl.BlockSpec((D, D), lambda g: (0, 0)),
            pl.BlockSpec((1, D), lambda g: (0, 0)),
        ],
        out_specs=pl.BlockSpec((G, D), lambda g: (0, 0)),
        out_shape=jax.ShapeDtypeStruct((G, D), jnp.float32),
        scratch_shapes=[
            pltpu.VMEM((G, D), jnp.float32),
            pltpu.VMEM((G, D), jnp.float32),
        ],
    )(batch.reshape(NPB, 1, BP), h, W_lin, b_lin.reshape(1, D))


def kernel(x, edge_index, edge_attr, batch,
           W_edge1, b_edge1, W_mlp1_1, bn_mlp_scale1, bn_mlp_bias1, W_mlp2_1, norm_scale1, norm_bias1,
           W_edge2, b_edge2, W_mlp1_2, bn_mlp_scale2, bn_mlp_bias2, W_mlp2_2, norm_scale2, norm_bias2,
           W_edge3, b_edge3, W_mlp1_3, bn_mlp_scale3, bn_mlp_bias3, W_mlp2_3, norm_scale3, norm_bias3,
           W_lin, b_lin):
    src4 = edge_index[0].reshape(NS, NBK, BK, CH)
    dst4 = edge_index[1].reshape(NS, NBK, BK, CH)
    params = [
        (W_edge1, b_edge1, W_mlp1_1, bn_mlp_scale1, bn_mlp_bias1, W_mlp2_1, norm_scale1, norm_bias1),
        (W_edge2, b_edge2, W_mlp1_2, bn_mlp_scale2, bn_mlp_bias2, W_mlp2_2, norm_scale2, norm_bias2),
        (W_edge3, b_edge3, W_mlp1_3, bn_mlp_scale3, bn_mlp_bias3, W_mlp2_3, norm_scale3, norm_bias3),
    ]
    xn = x
    for (We, be, W1, bs, bb, W2, ns, nb) in params:
        es = _edge_mlp(edge_attr, We, be)
        a = _sc_edge_pass(xn, es, src4, dst4)
        h1, s1, q1 = _aggr_mlp1(a, xn, W1)
        h2, s2, q2 = _bn_mlp2(h1, s1, q1, bs, bb, W2)
        xn = _bn_out(h2, s2, q2, ns, nb)
    return _pool_linear(batch, xn, W_lin, b_lin)


# speedup vs baseline: 1.0002x; 1.0002x over previous
import functools

import jax
import jax.numpy as jnp
from jax import lax
from jax.experimental import pallas as pl
from jax.experimental.pallas import tpu as pltpu
from jax.experimental.pallas import tpu_sc as plsc

N = 10000
E = 320000
D = 128
HD = 64
G = 64
EPS = 1e-7

NC = 2
NS = 16
CH = 40
EPT = E // NS
NCHUNK = EPT // CH
SR = 640

_mesh = plsc.VectorSubcoreMesh(core_axis_name="c", subcore_axis_name="s")


BK = 20
NBK = NCHUNK // BK


@functools.partial(
    pl.kernel,
    out_type=jax.ShapeDtypeStruct((2 * N, D), jnp.float32),
    mesh=_mesh,
    scratch_types=[
        pltpu.VMEM((BK, CH), jnp.int32),
        pltpu.VMEM((BK, CH), jnp.int32),
        pltpu.VMEM((CH, D), jnp.float32),
        pltpu.VMEM((CH, D), jnp.float32),
        pltpu.VMEM((CH, D), jnp.float32),
        pltpu.VMEM((CH, D), jnp.float32),
        pltpu.VMEM((CH, D), jnp.float32),
        pltpu.VMEM_SHARED((N, D), jnp.float32),
        pltpu.SemaphoreType.DMA,
        pltpu.SemaphoreType.DMA,
    ],
)
def _sc_edge_pass(xs_hbm, es_hbm, src_hbm, dst_hbm, a_out,
                  srcb, dstb, xv0, xv1, ev0, ev1, pq, acc_sh, sg, se):
    c = lax.axis_index("c")
    s = lax.axis_index("s")
    coffx = c * N
    xvs, evs = (xv0, xv1), (ev0, ev1)

    def zrow(i, carry):
        for cc in range(D // 16):
            pq[i, pl.ds(cc * 16, 16)] = jnp.zeros((16,), jnp.float32)
        return carry
    lax.fori_loop(0, CH, zrow, 0)
    stripe0 = s * SR

    @pl.when(s < NS - 1)
    def _():
        for k in range(SR // CH):
            r0 = pl.multiple_of(stripe0 + k * CH, 8)
            pltpu.sync_copy(pq, acc_sh.at[pl.ds(r0, CH)])

    @pl.when(s == NS - 1)
    def _():
        for k in range((N - (NS - 1) * SR) // CH):
            r0 = pl.multiple_of(stripe0 + k * CH, 8)
            pltpu.sync_copy(pq, acc_sh.at[pl.ds(r0, CH)])

    plsc.subcore_barrier()

    e0 = s * EPT

    def _ebase(g):
        return pl.ds(pl.multiple_of(e0 + g * CH, 8), CH)

    def _compute_scatter(xv, ev, drow):
        def crow(r8, carry3):
            for rr in range(8):
                r = r8 * 8 + rr
                for cc in range(HD // 16):
                    sl = pl.ds(c * HD + cc * 16, 16)
                    m = jnp.maximum(xv[r, sl] + ev[r, sl], 0.0) + EPS
                    p = jnp.exp(m)
                    pq[r, pl.ds(cc * 16, 16)] = p
                    pq[r, pl.ds(HD + cc * 16, 16)] = p * m
            return carry3
        lax.fori_loop(0, CH // 8, crow, 0)
        pltpu.sync_copy(pq, acc_sh.at[drow], add=True)

    def block(j, carry):
        @pl.when(j >= 1)
        def _():
            _compute_scatter(xv1, ev1, dstb.at[BK - 1])

        pltpu.sync_copy(src_hbm.at[s, j], srcb)
        pltpu.sync_copy(dst_hbm.at[s, j], dstb)

        def pair(ci, carry2):
            for b in range(2):
                i_loc = ci * 2 + b
                g = j * BK + i_loc
                d1 = pltpu.async_copy(xs_hbm.at[srcb.at[i_loc]], xvs[b], sg)
                d2 = pltpu.async_copy(es_hbm.at[_ebase(g)], evs[b], se)

                @pl.when(i_loc >= 1)
                def _():
                    _compute_scatter(xvs[1 - b], evs[1 - b],
                                     dstb.at[i_loc - 1])

                d1.wait()
                d2.wait()
            return carry2
        lax.fori_loop(0, BK // 2, pair, 0)
        return carry
    lax.fori_loop(0, NBK, block, 0)

    _compute_scatter(xv1, ev1, dstb.at[BK - 1])
    plsc.subcore_barrier()

    @pl.when(s < NS - 1)
    def _():
        for k in range(SR // CH):
            r0 = pl.multiple_of(stripe0 + k * CH, 8)
            pltpu.sync_copy(acc_sh.at[pl.ds(r0, CH)],
                            a_out.at[pl.ds(pl.multiple_of(coffx + r0, 8), CH)])

    @pl.when(s == NS - 1)
    def _():
        for k in range((N - (NS - 1) * SR) // CH):
            r0 = pl.multiple_of(stripe0 + k * CH, 8)
            pltpu.sync_copy(acc_sh.at[pl.ds(r0, CH)],
                            a_out.at[pl.ds(pl.multiple_of(coffx + r0, 8), CH)])


BE = 4000
NEB = E // BE
BN = 2000
NB = N // BN


def _edge_mlp_body(ea_ref, w_ref, b_ref, o_ref):
    o_ref[...] = (
        jnp.dot(ea_ref[...], w_ref[...], preferred_element_type=jnp.float32)
        + b_ref[...]
    )


def _edge_mlp(edge_attr, W_edge, b_edge):
    return pl.pallas_call(
        _edge_mlp_body,
        grid=(NEB,),
        in_specs=[
            pl.BlockSpec((BE, 16), lambda g: (g, 0)),
            pl.BlockSpec((16, D), lambda g: (0, 0)),
            pl.BlockSpec((1, D), lambda g: (0, 0)),
        ],
        out_specs=pl.BlockSpec((BE, D), lambda g: (g, 0)),
        out_shape=jax.ShapeDtypeStruct((E, D), jnp.float32),
    )(edge_attr, W_edge, b_edge.reshape(1, D))


def _aggr_mlp1_body(alo_ref, ahi_ref, x_ref, w1_ref, h1_ref, sum_ref, sq_ref):
    alo = alo_ref[...]
    ahi = ahi_ref[...]
    aggr = jnp.concatenate(
        [alo[:, HD:] / (alo[:, :HD] + 1e-16), ahi[:, HD:] / (ahi[:, :HD] + 1e-16)],
        axis=1,
    )
    out = aggr + x_ref[...]
    h1 = jnp.dot(out, w1_ref[...], preferred_element_type=jnp.float32)
    h1_ref[...] = h1

    @pl.when(pl.program_id(0) == 0)
    def _():
        sum_ref[...] = jnp.zeros_like(sum_ref)
        sq_ref[...] = jnp.zeros_like(sq_ref)

    sum_ref[...] += jnp.sum(h1, 0, keepdims=True)
    sq_ref[...] += jnp.sum(h1 * h1, 0, keepdims=True)


def _aggr_mlp1(a, x, W1):
    return pl.pallas_call(
        _aggr_mlp1_body,
        grid=(NB,),
        in_specs=[
            pl.BlockSpec((BN, D), lambda g: (g, 0)),
            pl.BlockSpec((BN, D), lambda g: (NB + g, 0)),
            pl.BlockSpec((BN, D), lambda g: (g, 0)),
            pl.BlockSpec((D, 2 * D), lambda g: (0, 0)),
        ],
        out_specs=[
            pl.BlockSpec((BN, 2 * D), lambda g: (g, 0)),
            pl.BlockSpec((1, 2 * D), lambda g: (0, 0)),
            pl.BlockSpec((1, 2 * D), lambda g: (0, 0)),
        ],
        out_shape=[
            jax.ShapeDtypeStruct((N, 2 * D), jnp.float32),
            jax.ShapeDtypeStruct((1, 2 * D), jnp.float32),
            jax.ShapeDtypeStruct((1, 2 * D), jnp.float32),
        ],
    )(a, a, x, W1)


def _bn_mlp2_body(h1_ref, sum_ref, sq_ref, bs_ref, bb_ref, w2_ref,
                  h2_ref, sum2_ref, sq2_ref):
    mu = sum_ref[...] / N
    var = sq_ref[...] / N - mu * mu
    inv = lax.rsqrt(var + 1e-5) * bs_ref[...]
    hn = jnp.maximum((h1_ref[...] - mu) * inv + bb_ref[...], 0.0)
    h2 = jnp.dot(hn, w2_ref[...], preferred_element_type=jnp.float32)
    h2_ref[...] = h2

    @pl.when(pl.program_id(0) == 0)
    def _():
        sum2_ref[...] = jnp.zeros_like(sum2_ref)
        sq2_ref[...] = jnp.zeros_like(sq2_ref)

    sum2_ref[...] += jnp.sum(h2, 0, keepdims=True)
    sq2_ref[...] += jnp.sum(h2 * h2, 0, keepdims=True)


def _bn_mlp2(h1, s1, q1, bs, bb, W2):
    return pl.pallas_call(
        _bn_mlp2_body,
        grid=(NB,),
        in_specs=[
            pl.BlockSpec((BN, 2 * D), lambda g: (g, 0)),
            pl.BlockSpec((1, 2 * D), lambda g: (0, 0)),
            pl.BlockSpec((1, 2 * D), lambda g: (0, 0)),
            pl.BlockSpec((1, 2 * D), lambda g: (0, 0)),
            pl.BlockSpec((1, 2 * D), lambda g: (0, 0)),
            pl.BlockSpec((2 * D, D), lambda g: (0, 0)),
        ],
        out_specs=[
            pl.BlockSpec((BN, D), lambda g: (g, 0)),
            pl.BlockSpec((1, D), lambda g: (0, 0)),
            pl.BlockSpec((1, D), lambda g: (0, 0)),
        ],
        out_shape=[
            jax.ShapeDtypeStruct((N, D), jnp.float32),
            jax.ShapeDtypeStruct((1, D), jnp.float32),
            jax.ShapeDtypeStruct((1, D), jnp.float32),
        ],
    )(h1, s1, q1, bs.reshape(1, 2 * D), bb.reshape(1, 2 * D), W2)


def _bn_out_body(h2_ref, sum_ref, sq_ref, ns_ref, nb_ref, xn_ref):
    mu = sum_ref[...] / N
    var = sq_ref[...] / N - mu * mu
    inv = lax.rsqrt(var + 1e-5) * ns_ref[...]
    xn_ref[...] = jnp.maximum((h2_ref[...] - mu) * inv + nb_ref[...], 0.0)


def _bn_out(h2, s2, q2, ns, nb):
    return pl.pallas_call(
        _bn_out_body,
        grid=(NB,),
        in_specs=[
            pl.BlockSpec((BN, D), lambda g: (g, 0)),
            pl.BlockSpec((1, D), lambda g: (0, 0)),
            pl.BlockSpec((1, D), lambda g: (0, 0)),
            pl.BlockSpec((1, D), lambda g: (0, 0)),
            pl.BlockSpec((1, D), lambda g: (0, 0)),
        ],
        out_specs=pl.BlockSpec((BN, D), lambda g: (g, 0)),
        out_shape=jax.ShapeDtypeStruct((N, D), jnp.float32),
    )(h2, s2, q2, ns.reshape(1, D), nb.reshape(1, D))


BP = 80
NPB = N // BP


def _pool_body(b_ref, x_ref, wl_ref, bl_ref, o_ref, acc, cnt):
    g = pl.program_id(0)

    @pl.when(g == 0)
    def _():
        acc[...] = jnp.zeros_like(acc)
        cnt[...] = jnp.zeros_like(cnt)

    oh_t = (
        lax.broadcasted_iota(jnp.int32, (G, BP), 0) == b_ref[0, 0, :][None, :]
    ).astype(jnp.float32)
    acc[...] += jnp.dot(oh_t, x_ref[...], preferred_element_type=jnp.float32)
    cnt[...] += jnp.dot(oh_t, jnp.ones((BP, D), jnp.float32),
                        preferred_element_type=jnp.float32)

    @pl.when(g == NPB - 1)
    def _():
        pooled = acc[...] / jnp.maximum(cnt[...], 1.0)
        o_ref[...] = (
            jnp.dot(pooled, wl_ref[...], preferred_element_type=jnp.float32)
            + bl_ref[...]
        )


def _pool_linear(batch, h, W_lin, b_lin):
    return pl.pallas_call(
        _pool_body,
        grid=(NPB,),
        in_specs=[
            pl.BlockSpec((1, 1, BP), lambda g: (g, 0, 0)),
            pl.BlockSpec((BP, D), lambda g: (g, 0)),
            pl.BlockSpec((D, D), lambda g: (0, 0)),
            pl.BlockSpec((1, D), lambda g: (0, 0)),
        ],
        out_specs=pl.BlockSpec((G, D), lambda g: (0, 0)),
        out_shape=jax.ShapeDtypeStruct((G, D), jnp.float32),
        scratch_shapes=[
            pltpu.VMEM((G, D), jnp.float32),
            pltpu.VMEM((G, D), jnp.float32),
        ],
    )(batch.reshape(NPB, 1, BP), h, W_lin, b_lin.reshape(1, D))


def kernel(x, edge_index, edge_attr, batch,
           W_edge1, b_edge1, W_mlp1_1, bn_mlp_scale1, bn_mlp_bias1, W_mlp2_1, norm_scale1, norm_bias1,
           W_edge2, b_edge2, W_mlp1_2, bn_mlp_scale2, bn_mlp_bias2, W_mlp2_2, norm_scale2, norm_bias2,
           W_edge3, b_edge3, W_mlp1_3, bn_mlp_scale3, bn_mlp_bias3, W_mlp2_3, norm_scale3, norm_bias3,
           W_lin, b_lin):
    src4 = edge_index[0].reshape(NS, NBK, BK, CH)
    dst4 = edge_index[1].reshape(NS, NBK, BK, CH)
    params = [
        (W_edge1, b_edge1, W_mlp1_1, bn_mlp_scale1, bn_mlp_bias1, W_mlp2_1, norm_scale1, norm_bias1),
        (W_edge2, b_edge2, W_mlp1_2, bn_mlp_scale2, bn_mlp_bias2, W_mlp2_2, norm_scale2, norm_bias2),
        (W_edge3, b_edge3, W_mlp1_3, bn_mlp_scale3, bn_mlp_bias3, W_mlp2_3, norm_scale3, norm_bias3),
    ]
    xn = x
    for (We, be, W1, bs, bb, W2, ns, nb) in params:
        es = _edge_mlp(edge_attr, We, be)
        a = _sc_edge_pass(xn, es, src4, dst4)
        h1, s1, q1 = _aggr_mlp1(a, xn, W1)
        h2, s2, q2 = _bn_mlp2(h1, s1, q1, bs, bb, W2)
        xn = _bn_out(h2, s2, q2, ns, nb)
    return _pool_linear(batch, xn, W_lin, b_lin)
